# baseline (speedup 1.0000x reference)
import functools

import jax
import jax.numpy as jnp
from jax import lax
from jax.experimental import pallas as pl
from jax.experimental.pallas import tpu as pltpu
from jax.experimental.pallas import tpu_sc as plsc

E = 8
K = 2
CAP = 512
NEG_INF = -1e30
NC = 2
NS = 16
LN = 16


def _router_body(x_ref, gw_ref, mslotT_ref, pT_ref, fsT_ref, aux_ref,
                 used_ref):
    x = x_ref[...]
    gw = gw_ref[...]
    T = x.shape[0]
    logits = jax.lax.dot_general(
        x, gw, (((1,), (1,)), ((), ())), preferred_element_type=jnp.float32)
    lane = jax.lax.broadcasted_iota(jnp.int32, (T, E), 1)
    m1 = jnp.max(logits, axis=1, keepdims=True)
    idx1 = jnp.min(jnp.where(logits == m1, lane, E), axis=1, keepdims=True)
    masked = jnp.where(lane == idx1, NEG_INF, logits)
    m2 = jnp.max(masked, axis=1, keepdims=True)
    idx2 = jnp.min(jnp.where(masked == m2, lane, E), axis=1, keepdims=True)
    t = jnp.exp(m2 - m1)
    w1 = 1.0 / (1.0 + t)
    w2 = t / (1.0 + t)
    mask = jnp.logical_or(lane == idx1, lane == idx2).astype(jnp.float32)
    inc = mask
    shift = 1
    while shift < T:
        shifted = jnp.concatenate(
            [jnp.zeros((shift, E), jnp.float32), inc[:T - shift]], axis=0)
        inc = inc + shifted
        shift *= 2
    slot = inc - mask
    kept = jnp.logical_and(mask > 0, slot < float(CAP))
    mslot = jnp.where(kept, slot, -1.0)
    p = jnp.where(kept, jnp.where(lane == idx1, w1, w2), 0.0)
    mslotT_ref[...] = mslot.T
    pT_ref[...] = p.T

    is1 = lane == idx1
    is2 = lane == idx2
    slot1 = jnp.sum(jnp.where(is1, slot, 0.0), axis=1, keepdims=True)
    slot2 = jnp.sum(jnp.where(is2, slot, 0.0), axis=1, keepdims=True)
    e1 = idx1.astype(jnp.float32)
    e2 = idx2.astype(jnp.float32)
    fs1 = jnp.where(slot1 < float(CAP), e1 * float(CAP) + slot1, -1.0)
    fs2 = jnp.where(slot2 < float(CAP), e2 * float(CAP) + slot2, -1.0)
    fs_te = jnp.where(lane == 0, fs1, jnp.where(lane == 1, fs2, 0.0))
    fsT_ref[...] = fs_te.T

    counts = jnp.sum(mask, axis=0, keepdims=True)
    frac = counts / float(T)
    mu = jnp.sum(frac, axis=1, keepdims=True) / float(E)
    var = jnp.sum((frac - mu) ** 2, axis=1, keepdims=True) / float(E - 1)
    aux_ref[...] = jnp.broadcast_to(var * float(E), (1, 128))
    used = jnp.sum((counts > 0).astype(jnp.int32), axis=1, keepdims=True)
    used_ref[...] = jnp.broadcast_to(used, (1, 128))


def _dispatch_body(x_hbm, mslotT_hbm, pT_hbm, xin_hbm, wslot_hbm,
                   slotrow, prow, selidx, wslot, rows, sem, *, T, H, g):
    c = lax.axis_index("c")
    s = lax.axis_index("s")
    w = s * NC + c
    el = w // 8
    e = g * 4 + el
    part = w % 8
    pltpu.sync_copy(mslotT_hbm.at[e], slotrow)
    pltpu.sync_copy(pT_hbm.at[e], prow)
    zi = jnp.zeros((LN,), jnp.int32)
    zf = jnp.zeros((LN,), jnp.float32)
    for i in range(CAP // LN):
        selidx[pl.ds(i * LN, LN)] = zi
        wslot[pl.ds(i * LN, LN)] = zf

    def scan_body(i, carry):
        sl = slotrow[pl.ds(i * LN, LN)]
        pv = prow[pl.ds(i * LN, LN)]
        sli = sl.astype(jnp.int32)
        msk = sl >= 0.0
        tok = lax.iota(jnp.int32, LN) + i * LN
        plsc.store_scatter(selidx, [sli], tok, mask=msk)
        plsc.store_scatter(wslot, [sli], pv, mask=msk)
        return carry

    lax.fori_loop(0, T // LN, scan_body, 0)
    off = part * 64
    pltpu.sync_copy(wslot.at[pl.ds(off, 64)],
                    wslot_hbm.at[pl.ds(el * CAP + off, 64)])
    idxref = selidx.at[pl.ds(off, 64)]
    pltpu.async_copy(x_hbm.at[idxref], rows, sem).wait()
    pltpu.sync_copy(rows, xin_hbm.at[pl.ds(el * CAP + off, 64)])


def _ffn_body(xin_ref, ws_ref, wg_ref, wu_ref, wd_ref, eo_ref, xb_ref,
              acc_ref, *, n_itile, ne):
    e = pl.program_id(0)
    it = pl.program_id(1)

    @pl.when(jnp.logical_and(e < ne, it == 0))
    def _():
        xb_ref[...] = xin_ref[...].astype(jnp.bfloat16)

    @pl.when(e < ne)
    def _compute():
        xin = xb_ref[...]
        wg = wg_ref[0].astype(jnp.bfloat16)
        wu = wu_ref[0].astype(jnp.bfloat16)
        g = jax.lax.dot_general(xin, wg, (((1,), (1,)), ((), ())),
                                preferred_element_type=jnp.float32)
        u = jax.lax.dot_general(xin, wu, (((1,), (1,)), ((), ())),
                                preferred_element_type=jnp.float32)
        g = g / (1.0 + jnp.exp(-g))
        h = (g * u).astype(jnp.bfloat16)
        wd = wd_ref[0].astype(jnp.bfloat16)
        contrib = jax.lax.dot_general(h, wd, (((1,), (1,)), ((), ())),
                                      preferred_element_type=jnp.float32)

        @pl.when(it == 0)
        def _():
            acc_ref[...] = contrib

        @pl.when(it > 0)
        def _():
            acc_ref[...] = acc_ref[...] + contrib

    @pl.when(it == n_itile - 1)
    def _():
        acc = acc_ref[...] * ws_ref[0]
        eo_ref[...] = jnp.where(e < ne, acc, 0.0)


def _ffn_body_b(xin_ref, ws_ref, wg_ref, wu_ref, wd_ref, eo_in_ref, eo_ref,
                xb_ref, acc_ref, *, n_itile, ne):
    del eo_in_ref
    _ffn_body(xin_ref, ws_ref, wg_ref, wu_ref, wd_ref, eo_ref, xb_ref,
              acc_ref, n_itile=n_itile, ne=ne)


def _combine_body(eo_hbm, fsT_hbm, out_hbm, fs1v, fs2v, idx1, idx2,
                  rows1, rows2, outbuf, sem1, sem2, *, T, H):
    c = lax.axis_index("c")
    s = lax.axis_index("s")
    w = s * NC + c
    ntok = T // (NC * NS)
    tbase = w * ntok
    pltpu.sync_copy(fsT_hbm.at[0, pl.ds(tbase, ntok)], fs1v)
    pltpu.sync_copy(fsT_hbm.at[1, pl.ds(tbase, ntok)], fs2v)
    zrow = jnp.float32(E * CAP)
    for i in range(ntok // LN):
        sl = pl.ds(i * LN, LN)
        f1 = fs1v[sl]
        f2 = fs2v[sl]
        idx1[sl] = jnp.where(f1 >= 0.0, f1, zrow).astype(jnp.int32)
        idx2[sl] = jnp.where(f2 >= 0.0, f2, zrow).astype(jnp.int32)

    for ch in range(ntok // 32):
        cp1 = pltpu.async_copy(eo_hbm.at[idx1.at[pl.ds(ch * 32, 32)]],
                               rows1, sem1)
        cp2 = pltpu.async_copy(eo_hbm.at[idx2.at[pl.ds(ch * 32, 32)]],
                               rows2, sem2)
        cp1.wait()
        cp2.wait()

        def tok_body(i, carry):
            for j in range(H // LN):
                js = pl.ds(j * LN, LN)
                outbuf[i, js] = rows1[i, js] + rows2[i, js]
            return carry

        lax.fori_loop(0, 32, tok_body, 0)
        pltpu.sync_copy(outbuf, out_hbm.at[pl.ds(tbase + ch * 32, 32)])


def kernel(x, gate_w, w_gate, w_up, w_down):
    B, S, H = x.shape
    T = B * S
    I = w_gate.shape[1]
    x_flat = x.reshape(T, H)

    mslotT, pT, fsT, aux, used = pl.pallas_call(
        _router_body,
        out_shape=(
            jax.ShapeDtypeStruct((E, T), jnp.float32),
            jax.ShapeDtypeStruct((E, T), jnp.float32),
            jax.ShapeDtypeStruct((E, T), jnp.float32),
            jax.ShapeDtypeStruct((1, 128), jnp.float32),
            jax.ShapeDtypeStruct((1, 128), jnp.int32),
        ),
    )(x_flat, gate_w)

    mesh = plsc.VectorSubcoreMesh(core_axis_name="c", subcore_axis_name="s",
                                  num_cores=NC, num_subcores=NS)
    EH = E // 2

    def _dispatch_half(g):
        return pl.kernel(
            functools.partial(_dispatch_body, T=T, H=H, g=g),
            out_type=(
                jax.ShapeDtypeStruct((EH * CAP, H), jnp.float32),
                jax.ShapeDtypeStruct((EH * CAP,), jnp.float32),
            ),
            mesh=mesh,
            scratch_types=[
                pltpu.VMEM((T,), jnp.float32),
                pltpu.VMEM((T,), jnp.float32),
                pltpu.VMEM((CAP,), jnp.int32),
                pltpu.VMEM((CAP,), jnp.float32),
                pltpu.VMEM((64, H), jnp.float32),
                pltpu.SemaphoreType.DMA,
            ],
            compiler_params=pltpu.CompilerParams(needs_layout_passes=False),
        )(x_flat, mslotT, pT)

    xin_a, wslot_a = _dispatch_half(0)
    xin_b, wslot_b = _dispatch_half(1)

    IT = 1024
    n_itile = I // IT
    ws3_a = wslot_a.reshape(EH, CAP, 1)
    ws3_b = wslot_b.reshape(EH, CAP, 1)
    _cp = pltpu.CompilerParams(
        dimension_semantics=("arbitrary", "arbitrary"),
        vmem_limit_bytes=62 * 1024 * 1024,
    )
    _scratch = [
        pltpu.VMEM((CAP, H), jnp.bfloat16),
        pltpu.VMEM((CAP, H), jnp.float32),
    ]
    eo_a = pl.pallas_call(
        functools.partial(_ffn_body, n_itile=n_itile, ne=EH),
        grid=(EH + 1, n_itile),
        in_specs=[
            pl.BlockSpec((CAP, H), lambda e, it: (jnp.minimum(e, EH - 1), 0)),
            pl.BlockSpec((1, CAP, 1),
                         lambda e, it: (jnp.minimum(e, EH - 1), 0, 0)),
            pl.BlockSpec((1, IT, H),
                         lambda e, it: (jnp.minimum(e, EH - 1),
                                        jnp.where(e < EH, it, n_itile - 1),
                                        0)),
            pl.BlockSpec((1, IT, H),
                         lambda e, it: (jnp.minimum(e, EH - 1),
                                        jnp.where(e < EH, it, n_itile - 1),
                                        0)),
            pl.BlockSpec((1, H, IT),
                         lambda e, it: (jnp.minimum(e, EH - 1), 0,
                                        jnp.where(e < EH, it, n_itile - 1))),
        ],
        out_specs=pl.BlockSpec((CAP, H),
                               lambda e, it: (jnp.where(e < EH, e, E), 0)),
        out_shape=jax.ShapeDtypeStruct(((E + 1) * CAP, H), jnp.float32),
        scratch_shapes=_scratch,
        compiler_params=_cp,
    )(xin_a, ws3_a, w_gate, w_up, w_down)
    eo = pl.pallas_call(
        functools.partial(_ffn_body_b, n_itile=n_itile, ne=EH),
        grid=(EH, n_itile),
        in_specs=[
            pl.BlockSpec((CAP, H), lambda e, it: (e, 0)),
            pl.BlockSpec((1, CAP, 1), lambda e, it: (e, 0, 0)),
            pl.BlockSpec((1, IT, H), lambda e, it: (e + EH, it, 0)),
            pl.BlockSpec((1, IT, H), lambda e, it: (e + EH, it, 0)),
            pl.BlockSpec((1, H, IT), lambda e, it: (e + EH, 0, it)),
            pl.BlockSpec(memory_space=pl.ANY),
        ],
        out_specs=pl.BlockSpec((CAP, H), lambda e, it: (e + EH, 0)),
        out_shape=jax.ShapeDtypeStruct(((E + 1) * CAP, H), jnp.float32),
        scratch_shapes=_scratch,
        compiler_params=_cp,
        input_output_aliases={5: 0},
    )(xin_b, ws3_b, w_gate, w_up, w_down, eo_a)

    out = pl.kernel(
        functools.partial(_combine_body, T=T, H=H),
        out_type=jax.ShapeDtypeStruct((T, H), jnp.float32),
        mesh=mesh,
        scratch_types=[
            pltpu.VMEM((T // (NC * NS),), jnp.float32),
            pltpu.VMEM((T // (NC * NS),), jnp.float32),
            pltpu.VMEM((T // (NC * NS),), jnp.int32),
            pltpu.VMEM((T // (NC * NS),), jnp.int32),
            pltpu.VMEM((32, H), jnp.float32),
            pltpu.VMEM((32, H), jnp.float32),
            pltpu.VMEM((32, H), jnp.float32),
            pltpu.SemaphoreType.DMA,
            pltpu.SemaphoreType.DMA,
        ],
        compiler_params=pltpu.CompilerParams(needs_layout_passes=False),
    )(eo, fsT)

    return (out.reshape(B, S, H), aux[0, 0], used[0, 0])


# speedup vs baseline: 1.0648x; 1.0648x over previous
import functools

import jax
import jax.numpy as jnp
from jax import lax
from jax.experimental import pallas as pl
from jax.experimental.pallas import tpu as pltpu
from jax.experimental.pallas import tpu_sc as plsc

E = 8
K = 2
CAP = 512
NEG_INF = -1e30
NC = 2
NS = 16
LN = 16


def _router_body(x_ref, gw_ref, mslotT_ref, pT_ref, fsT_ref, aux_ref,
                 used_ref):
    x = x_ref[...]
    gw = gw_ref[...]
    T = x.shape[0]
    logits = jax.lax.dot_general(
        x, gw, (((1,), (1,)), ((), ())), preferred_element_type=jnp.float32)
    lane = jax.lax.broadcasted_iota(jnp.int32, (T, E), 1)
    m1 = jnp.max(logits, axis=1, keepdims=True)
    idx1 = jnp.min(jnp.where(logits == m1, lane, E), axis=1, keepdims=True)
    masked = jnp.where(lane == idx1, NEG_INF, logits)
    m2 = jnp.max(masked, axis=1, keepdims=True)
    idx2 = jnp.min(jnp.where(masked == m2, lane, E), axis=1, keepdims=True)
    t = jnp.exp(m2 - m1)
    w1 = 1.0 / (1.0 + t)
    w2 = t / (1.0 + t)
    mask = jnp.logical_or(lane == idx1, lane == idx2).astype(jnp.float32)
    inc = mask
    shift = 1
    while shift < T:
        shifted = jnp.concatenate(
            [jnp.zeros((shift, E), jnp.float32), inc[:T - shift]], axis=0)
        inc = inc + shifted
        shift *= 2
    slot = inc - mask
    kept = jnp.logical_and(mask > 0, slot < float(CAP))
    mslot = jnp.where(kept, slot, -1.0)
    p = jnp.where(kept, jnp.where(lane == idx1, w1, w2), 0.0)
    mslotT_ref[...] = mslot.T
    pT_ref[...] = p.T

    is1 = lane == idx1
    is2 = lane == idx2
    slot1 = jnp.sum(jnp.where(is1, slot, 0.0), axis=1, keepdims=True)
    slot2 = jnp.sum(jnp.where(is2, slot, 0.0), axis=1, keepdims=True)
    e1 = idx1.astype(jnp.float32)
    e2 = idx2.astype(jnp.float32)
    fs1 = jnp.where(slot1 < float(CAP), e1 * float(CAP) + slot1, -1.0)
    fs2 = jnp.where(slot2 < float(CAP), e2 * float(CAP) + slot2, -1.0)
    fs_te = jnp.where(lane == 0, fs1, jnp.where(lane == 1, fs2, 0.0))
    fsT_ref[...] = fs_te.T

    counts = jnp.sum(mask, axis=0, keepdims=True)
    frac = counts / float(T)
    mu = jnp.sum(frac, axis=1, keepdims=True) / float(E)
    var = jnp.sum((frac - mu) ** 2, axis=1, keepdims=True) / float(E - 1)
    aux_ref[...] = jnp.broadcast_to(var * float(E), (1, 128))
    used = jnp.sum((counts > 0).astype(jnp.int32), axis=1, keepdims=True)
    used_ref[...] = jnp.broadcast_to(used, (1, 128))


def _dispatch_body(x_hbm, mslotT_hbm, pT_hbm, xin_hbm, wslot_hbm,
                   slotrow, prow, selidx, wslot, rows0, rows1,
                   gsem0, gsem1, wsem0, wsem1, *, T, H):
    c = lax.axis_index("c")
    s = lax.axis_index("s")
    e = 4 * c + s // 4
    q = s % 4
    pltpu.sync_copy(mslotT_hbm.at[e], slotrow)
    pltpu.sync_copy(pT_hbm.at[e], prow)
    zi = jnp.zeros((LN,), jnp.int32)
    zf = jnp.zeros((LN,), jnp.float32)
    for i in range(CAP // LN):
        selidx[pl.ds(i * LN, LN)] = zi
        wslot[pl.ds(i * LN, LN)] = zf

    def scan_body(i, carry):
        sl = slotrow[pl.ds(i * LN, LN)]
        pv = prow[pl.ds(i * LN, LN)]
        sli = sl.astype(jnp.int32)
        msk = sl >= 0.0
        tok = lax.iota(jnp.int32, LN) + i * LN
        plsc.store_scatter(selidx, [sli], tok, mask=msk)
        plsc.store_scatter(wslot, [sli], pv, mask=msk)
        return carry

    lax.fori_loop(0, T // LN, scan_body, 0)
    base = q * (CAP // 4)
    pltpu.sync_copy(wslot.at[pl.ds(base, CAP // 4)],
                    wslot_hbm.at[pl.ds(e * CAP + base, CAP // 4)])
    rows = (rows0, rows1)
    gsem = (gsem0, gsem1)
    wsem = (wsem0, wsem1)
    CH = 32

    def _fire_gather(ch):
        b = ch % 2
        off = base + ch * CH
        return pltpu.async_copy(x_hbm.at[selidx.at[pl.ds(off, CH)]],
                                rows[b], gsem[b])

    def _fire_write(ch):
        b = ch % 2
        off = base + ch * CH
        return pltpu.async_copy(rows[b],
                                xin_hbm.at[pl.ds(e * CAP + off, CH)],
                                wsem[b])

    g0 = _fire_gather(0)
    g1 = _fire_gather(1)
    g0.wait()
    w0 = _fire_write(0)
    g1.wait()
    w1 = _fire_write(1)
    w0.wait()
    g2 = _fire_gather(2)
    w1.wait()
    g3 = _fire_gather(3)
    g2.wait()
    w2 = _fire_write(2)
    g3.wait()
    w3 = _fire_write(3)
    w2.wait()
    w3.wait()


def _ffn_body(xin_ref, ws_ref, wg_ref, wu_ref, wd_ref, eo_ref, xb_ref,
              acc_ref, *, n_itile):
    e = pl.program_id(0)
    it = pl.program_id(1)

    @pl.when(jnp.logical_and(e < E, it == 0))
    def _():
        xb_ref[...] = xin_ref[...].astype(jnp.bfloat16)

    @pl.when(e < E)
    def _compute():
        xin = xb_ref[...]
        wg = wg_ref[0].astype(jnp.bfloat16)
        wu = wu_ref[0].astype(jnp.bfloat16)
        g = jax.lax.dot_general(xin, wg, (((1,), (1,)), ((), ())),
                                preferred_element_type=jnp.float32)
        u = jax.lax.dot_general(xin, wu, (((1,), (1,)), ((), ())),
                                preferred_element_type=jnp.float32)
        g = g / (1.0 + jnp.exp(-g))
        h = (g * u).astype(jnp.bfloat16)
        wd = wd_ref[0].astype(jnp.bfloat16)
        contrib = jax.lax.dot_general(h, wd, (((1,), (1,)), ((), ())),
                                      preferred_element_type=jnp.float32)

        @pl.when(it == 0)
        def _():
            acc_ref[...] = contrib

        @pl.when(it > 0)
        def _():
            acc_ref[...] = acc_ref[...] + contrib

    @pl.when(it == n_itile - 1)
    def _():
        acc = acc_ref[...] * ws_ref[0]
        eo_ref[...] = jnp.where(e < E, acc, 0.0)


def _combine_body(eo_hbm, fsT_hbm, out_hbm, fs1v, fs2v, idx1, idx2,
                  rows1a, rows1b, rows2a, rows2b, outa, outb_,
                  g1a, g1b, g2a, g2b, wsa, wsb, *, T, H):
    c = lax.axis_index("c")
    s = lax.axis_index("s")
    w = s * NC + c
    ntok = T // (NC * NS)
    tbase = w * ntok
    pltpu.sync_copy(fsT_hbm.at[0, pl.ds(tbase, ntok)], fs1v)
    pltpu.sync_copy(fsT_hbm.at[1, pl.ds(tbase, ntok)], fs2v)
    zrow = jnp.float32(E * CAP)
    for i in range(ntok // LN):
        sl = pl.ds(i * LN, LN)
        f1 = fs1v[sl]
        f2 = fs2v[sl]
        idx1[sl] = jnp.where(f1 >= 0.0, f1, zrow).astype(jnp.int32)
        idx2[sl] = jnp.where(f2 >= 0.0, f2, zrow).astype(jnp.int32)

    CH = 16
    nch = ntok // CH
    rows1 = (rows1a, rows1b)
    rows2 = (rows2a, rows2b)
    outb = (outa, outb_)
    g1s = (g1a, g1b)
    g2s = (g2a, g2b)
    wss = (wsa, wsb)

    def _fire(ch):
        b = ch % 2
        c1 = pltpu.async_copy(eo_hbm.at[idx1.at[pl.ds(ch * CH, CH)]],
                              rows1[b], g1s[b])
        c2 = pltpu.async_copy(eo_hbm.at[idx2.at[pl.ds(ch * CH, CH)]],
                              rows2[b], g2s[b])
        return c1, c2

    pend_g = [None, None]
    pend_w = [None, None]
    pend_g[0] = _fire(0)
    for ch in range(nch):
        b = ch % 2
        nb = 1 - b
        if ch + 1 < nch:
            if pend_w[nb] is not None:
                pend_w[nb].wait()
                pend_w[nb] = None
            pend_g[nb] = _fire(ch + 1)
        pend_g[b][0].wait()
        pend_g[b][1].wait()
        if pend_w[b] is not None:
            pend_w[b].wait()
            pend_w[b] = None
        r1 = rows1[b]
        r2 = rows2[b]
        ob = outb[b]

        def tok_body(i, carry):
            for j in range(H // LN):
                js = pl.ds(j * LN, LN)
                ob[i, js] = r1[i, js] + r2[i, js]
            return carry

        lax.fori_loop(0, CH, tok_body, 0)
        pend_w[b] = pltpu.async_copy(
            ob, out_hbm.at[pl.ds(tbase + ch * CH, CH)], wss[b])
    for b in range(2):
        if pend_w[b] is not None:
            pend_w[b].wait()


def kernel(x, gate_w, w_gate, w_up, w_down):
    B, S, H = x.shape
    T = B * S
    I = w_gate.shape[1]
    x_flat = x.reshape(T, H)

    mslotT, pT, fsT, aux, used = pl.pallas_call(
        _router_body,
        out_shape=(
            jax.ShapeDtypeStruct((E, T), jnp.float32),
            jax.ShapeDtypeStruct((E, T), jnp.float32),
            jax.ShapeDtypeStruct((E, T), jnp.float32),
            jax.ShapeDtypeStruct((1, 128), jnp.float32),
            jax.ShapeDtypeStruct((1, 128), jnp.int32),
        ),
    )(x_flat, gate_w)

    mesh = plsc.VectorSubcoreMesh(core_axis_name="c", subcore_axis_name="s",
                                  num_cores=NC, num_subcores=NS)
    xin, wslot = pl.kernel(
        functools.partial(_dispatch_body, T=T, H=H),
        out_type=(
            jax.ShapeDtypeStruct((E * CAP, H), jnp.float32),
            jax.ShapeDtypeStruct((E * CAP,), jnp.float32),
        ),
        mesh=mesh,
        scratch_types=[
            pltpu.VMEM((T,), jnp.float32),
            pltpu.VMEM((T,), jnp.float32),
            pltpu.VMEM((CAP,), jnp.int32),
            pltpu.VMEM((CAP,), jnp.float32),
            pltpu.VMEM((32, H), jnp.float32),
            pltpu.VMEM((32, H), jnp.float32),
            pltpu.SemaphoreType.DMA,
            pltpu.SemaphoreType.DMA,
            pltpu.SemaphoreType.DMA,
            pltpu.SemaphoreType.DMA,
        ],
        compiler_params=pltpu.CompilerParams(needs_layout_passes=False),
    )(x_flat, mslotT, pT)

    IT = 1024
    n_itile = I // IT
    wslot3 = wslot.reshape(E, CAP, 1)
    eo = pl.pallas_call(
        functools.partial(_ffn_body, n_itile=n_itile),
        grid=(E + 1, n_itile),
        in_specs=[
            pl.BlockSpec((CAP, H), lambda e, it: (jnp.minimum(e, E - 1), 0)),
            pl.BlockSpec((1, CAP, 1),
                         lambda e, it: (jnp.minimum(e, E - 1), 0, 0)),
            pl.BlockSpec((1, IT, H),
                         lambda e, it: (jnp.minimum(e, E - 1),
                                        jnp.where(e < E, it, n_itile - 1), 0)),
            pl.BlockSpec((1, IT, H),
                         lambda e, it: (jnp.minimum(e, E - 1),
                                        jnp.where(e < E, it, n_itile - 1), 0)),
            pl.BlockSpec((1, H, IT),
                         lambda e, it: (jnp.minimum(e, E - 1), 0,
                                        jnp.where(e < E, it, n_itile - 1))),
        ],
        out_specs=pl.BlockSpec((CAP, H), lambda e, it: (e, 0)),
        out_shape=jax.ShapeDtypeStruct(((E + 1) * CAP, H), jnp.float32),
        scratch_shapes=[
            pltpu.VMEM((CAP, H), jnp.bfloat16),
            pltpu.VMEM((CAP, H), jnp.float32),
        ],
        compiler_params=pltpu.CompilerParams(
            dimension_semantics=("arbitrary", "arbitrary"),
            vmem_limit_bytes=62 * 1024 * 1024,
        ),
    )(xin, wslot3, w_gate, w_up, w_down)

    out = pl.kernel(
        functools.partial(_combine_body, T=T, H=H),
        out_type=jax.ShapeDtypeStruct((T, H), jnp.float32),
        mesh=mesh,
        scratch_types=[
            pltpu.VMEM((T // (NC * NS),), jnp.float32),
            pltpu.VMEM((T // (NC * NS),), jnp.float32),
            pltpu.VMEM((T // (NC * NS),), jnp.int32),
            pltpu.VMEM((T // (NC * NS),), jnp.int32),
            pltpu.VMEM((16, H), jnp.float32),
            pltpu.VMEM((16, H), jnp.float32),
            pltpu.VMEM((16, H), jnp.float32),
            pltpu.VMEM((16, H), jnp.float32),
            pltpu.VMEM((16, H), jnp.float32),
            pltpu.VMEM((16, H), jnp.float32),
            pltpu.SemaphoreType.DMA,
            pltpu.SemaphoreType.DMA,
            pltpu.SemaphoreType.DMA,
            pltpu.SemaphoreType.DMA,
            pltpu.SemaphoreType.DMA,
            pltpu.SemaphoreType.DMA,
        ],
        compiler_params=pltpu.CompilerParams(needs_layout_passes=False),
    )(eo, fsT)

    return (out.reshape(B, S, H), aux[0, 0], used[0, 0])
